# baseline (speedup 1.0000x reference)
import jax
import jax.numpy as jnp
from jax.experimental import pallas as pl
from jax.experimental.pallas import tpu as pltpu


def _chunk_bounds(seq_len):
    sizes = [seq_len * 3 // 8 // 8 * 8, 0]
    sizes[1] = seq_len - sizes[0]
    bounds, start = [], 0
    for size in sizes:
        bounds.append((start, size))
        start += size
    return bounds


_N_CHUNKS = len(_chunk_bounds(8192))


def _copy_body(pos_hbm, out_hbm, buf, in_sems, out_sems):
    batch = out_hbm.shape[0]
    seq_len = pos_hbm.shape[0]
    bounds = _chunk_bounds(seq_len)

    def in_copy(c):
        rows = pl.ds(bounds[c][0], bounds[c][1])
        return pltpu.make_async_copy(pos_hbm.at[rows], buf.at[rows], in_sems.at[c])

    def out_copy(c, b):
        rows = pl.ds(bounds[c][0], bounds[c][1])
        return pltpu.make_async_copy(buf.at[rows], out_hbm.at[b, rows], out_sems.at[c, b])

    for c in range(_N_CHUNKS):
        in_copy(c).start()
    for c in range(_N_CHUNKS):
        in_copy(c).wait()
        for b in range(batch):
            out_copy(c, b).start()
    for c in range(_N_CHUNKS):
        for b in range(batch):
            out_copy(c, b).wait()


def kernel(x, pos_table):
    batch, seq_len = x.shape
    d_model = pos_table.shape[1]
    pos = pos_table[:seq_len]
    return pl.pallas_call(
        _copy_body,
        in_specs=[pl.BlockSpec(memory_space=pl.ANY)],
        out_specs=pl.BlockSpec(memory_space=pl.ANY),
        out_shape=jax.ShapeDtypeStruct((batch, seq_len, d_model), pos_table.dtype),
        scratch_shapes=[
            pltpu.VMEM((seq_len, d_model), pos_table.dtype),
            pltpu.SemaphoreType.DMA((_N_CHUNKS,)),
            pltpu.SemaphoreType.DMA((_N_CHUNKS, batch)),
        ],
    )(pos)


# speedup vs baseline: 1.0178x; 1.0178x over previous
import jax
import jax.numpy as jnp
from jax.experimental import pallas as pl
from jax.experimental.pallas import tpu as pltpu


def _chunk_bounds(seq_len):
    sizes = [seq_len // 2, seq_len - seq_len // 2]
    bounds, start = [], 0
    for size in sizes:
        bounds.append((start, size))
        start += size
    return bounds


_N_CHUNKS = len(_chunk_bounds(8192))


def _copy_body(pos_hbm, out_hbm, buf, in_sems, out_sems):
    batch = out_hbm.shape[0]
    seq_len = pos_hbm.shape[0]
    bounds = _chunk_bounds(seq_len)

    def in_copy(c):
        rows = pl.ds(bounds[c][0], bounds[c][1])
        return pltpu.make_async_copy(pos_hbm.at[rows], buf.at[rows], in_sems.at[c])

    def out_copy(c, b):
        rows = pl.ds(bounds[c][0], bounds[c][1])
        return pltpu.make_async_copy(buf.at[rows], out_hbm.at[b, rows], out_sems.at[c, b])

    for c in range(_N_CHUNKS):
        in_copy(c).start()
    for c in range(_N_CHUNKS):
        in_copy(c).wait()
        for b in range(batch):
            out_copy(c, b).start()
    for c in range(_N_CHUNKS):
        for b in range(batch):
            out_copy(c, b).wait()


def kernel(x, pos_table):
    batch, seq_len = x.shape
    d_model = pos_table.shape[1]
    pos = pos_table[:seq_len]
    return pl.pallas_call(
        _copy_body,
        in_specs=[pl.BlockSpec(memory_space=pl.ANY)],
        out_specs=pl.BlockSpec(memory_space=pl.ANY),
        out_shape=jax.ShapeDtypeStruct((batch, seq_len, d_model), pos_table.dtype),
        scratch_shapes=[
            pltpu.VMEM((seq_len, d_model), pos_table.dtype),
            pltpu.SemaphoreType.DMA((_N_CHUNKS,)),
            pltpu.SemaphoreType.DMA((_N_CHUNKS, batch)),
        ],
    )(pos)
